# baseline (speedup 1.0000x reference)
import jax
import jax.numpy as jnp
from jax import lax
from jax.experimental import pallas as pl
from jax.experimental.pallas import tpu as pltpu
from jax.experimental.pallas import tpu_sc as plsc

_B, _T, _C = 16, 32768, 64
_ROWS = _B * _T
_NC, _NS, _L = 2, 16, 16
_NW = _NC * _NS
_RPW = _ROWS // _NW
_CHR = 128
_NCH = _RPW // _CHR
_GRP = _CHR // _L
_RING = 2
_MCOL = _RPW // 8


def _sc_body(p_hbm, t_hbm, m_hbm, se_hbm, n_hbm,
             mb0, mb1, pb0, pb1, tb0, tb1,
             acc_v, cnt_v,
             sm0, sm1, sp0, sp1, st0, st1):
    wid = lax.axis_index("s") * _NC + lax.axis_index("c")
    b = wid // 2
    t0 = (wid % 2) * _RPW

    acc_v[...] = jnp.zeros((_L,), jnp.float32)
    cnt_v[...] = jnp.zeros((_L,), jnp.int32)

    ones = jnp.full((_L,), 1, jnp.int32)

    def fire(ch, mb, pb, tb, sm, sp, st):
        r0 = t0 + ch * _CHR
        mr = wid * 8 + ch // (_MCOL // _CHR)
        mc = (ch % (_MCOL // _CHR)) * _CHR
        pltpu.async_copy(m_hbm.at[mr, pl.ds(mc, _CHR)], mb, sm)
        pltpu.async_copy(p_hbm.at[b, pl.ds(r0, _CHR)], pb, sp)
        pltpu.async_copy(t_hbm.at[b, pl.ds(r0, _CHR)], tb, st)

    def wait(ch, mb, pb, tb, sm, sp, st):
        r0 = t0 + ch * _CHR
        mr = wid * 8 + ch // (_MCOL // _CHR)
        mc = (ch % (_MCOL // _CHR)) * _CHR
        pltpu.make_async_copy(m_hbm.at[mr, pl.ds(mc, _CHR)], mb, sm).wait()
        pltpu.make_async_copy(p_hbm.at[b, pl.ds(r0, _CHR)], pb, sp).wait()
        pltpu.make_async_copy(t_hbm.at[b, pl.ds(r0, _CHR)], tb, st).wait()

    def compute(ch, mb, pb, tb):
        acc = jnp.zeros((_L,), jnp.float32)
        for g in range(_GRP):
            m16 = mb[pl.ds(g * _L, _L)]
            cnt_v[...] = cnt_v[...] + m16
            m16f = m16.astype(jnp.float32)
            jvec = jnp.zeros((_L,), jnp.int32)
            for j in range(_L):
                msp = lax.gather(
                    m16f, jvec[:, None],
                    lax.GatherDimensionNumbers(
                        offset_dims=(), collapsed_slice_dims=(0,),
                        start_index_map=(0,)),
                    (1,), mode=lax.GatherScatterMode.PROMISE_IN_BOUNDS)
                r = g * _L + j
                for c4 in range(_C // _L):
                    pv = pb[r, pl.ds(c4 * _L, _L)]
                    tv = tb[r, pl.ds(c4 * _L, _L)]
                    d = pv - tv
                    acc = acc + d * d * msp
                jvec = jvec + ones
        acc_v[...] = acc_v[...] + acc

    bufs = ((mb0, pb0, tb0, sm0, sp0, st0), (mb1, pb1, tb1, sm1, sp1, st1))

    for r in range(_RING):
        fire(r, *bufs[r])

    def ring(kk, _):
        ch = _RING * kk
        for h in range(_RING):
            cc = ch + h
            wait(cc, *bufs[h])
            compute(cc, bufs[h][0], bufs[h][1], bufs[h][2])

            @pl.when(cc + _RING < _NCH)
            def _():
                fire(cc + _RING, *bufs[h])

        return 0

    lax.fori_loop(0, _NCH // _RING, ring, 0)

    pltpu.sync_copy(acc_v, se_hbm.at[wid])
    pltpu.sync_copy(cnt_v, n_hbm.at[wid])


def _sc_call(p3, t3, mi):
    kfn = pl.kernel(
        _sc_body,
        out_type=[
            jax.ShapeDtypeStruct((_NW, _L), jnp.float32),
            jax.ShapeDtypeStruct((_NW, _L), jnp.int32),
        ],
        mesh=plsc.VectorSubcoreMesh(core_axis_name="c", subcore_axis_name="s"),
        scratch_types=(
            [pltpu.VMEM((_CHR,), jnp.int32) for _ in range(_RING)]
            + [pltpu.VMEM((_CHR, _C), jnp.float32) for _ in range(2 * _RING)]
            + [pltpu.VMEM((_L,), jnp.float32), pltpu.VMEM((_L,), jnp.int32)]
            + [pltpu.SemaphoreType.DMA for _ in range(3 * _RING)]
        ),
        compiler_params=pltpu.CompilerParams(needs_layout_passes=False),
    )
    return kfn(p3, t3, mi)


def kernel(predicted, target, mask):
    mi = mask.astype(jnp.int32).reshape(_NW * 8, _MCOL)
    se, n = _sc_call(predicted, target, mi)
    se_tot = jnp.sum(se)
    n_tot = jnp.sum(n).astype(jnp.float32)
    count = n_tot * jnp.float32(_C)
    safe = jnp.where(count == 0.0, jnp.float32(1.0), count)
    return jnp.where(n_tot == 0.0, jnp.float32(0.0), se_tot / safe)


# speedup vs baseline: 1.1347x; 1.1347x over previous
import jax
import jax.numpy as jnp
from jax import lax
from jax.experimental import pallas as pl
from jax.experimental.pallas import tpu as pltpu
from jax.experimental.pallas import tpu_sc as plsc

_B, _T, _C = 16, 32768, 64
_BT = 8


_TB = 8192
_NQ = 4
_CH = _TB // _NQ
_SLABS_PER_B = _T // _TB
_STEPS = _BT * _SLABS_PER_B
_MR = _TB // 128


def _tc_body(p_hbm, t_hbm, m_ref, se_ref, n_ref,
             pb, tb, se_acc, n_acc, sp, st):
    i = pl.program_id(0)

    def fire(j, h):
        b = j // _SLABS_PER_B
        t0 = (j % _SLABS_PER_B) * _TB
        for q in range(_NQ):
            pltpu.make_async_copy(
                p_hbm.at[b, pl.ds(t0 + q * _CH, _CH)],
                pb.at[h, pl.ds(q * _CH, _CH)],
                sp.at[h, q],
            ).start()
            pltpu.make_async_copy(
                t_hbm.at[b, pl.ds(t0 + q * _CH, _CH)],
                tb.at[h, pl.ds(q * _CH, _CH)],
                st.at[h, q],
            ).start()

    def wait(j, h):
        b = j // _SLABS_PER_B
        t0 = (j % _SLABS_PER_B) * _TB
        for q in range(_NQ):
            pltpu.make_async_copy(
                p_hbm.at[b, pl.ds(t0 + q * _CH, _CH)],
                pb.at[h, pl.ds(q * _CH, _CH)],
                sp.at[h, q],
            ).wait()
            pltpu.make_async_copy(
                t_hbm.at[b, pl.ds(t0 + q * _CH, _CH)],
                tb.at[h, pl.ds(q * _CH, _CH)],
                st.at[h, q],
            ).wait()

    @pl.when(i == 0)
    def _():
        se_acc[0] = 0.0
        n_acc[0] = 0.0
        fire(0, 0)

    for h in (0, 1):
        @pl.when(((i + 1) % 2 == h) & (i + 1 < _STEPS))
        def _():
            fire(i + 1, h)

    for h in (0, 1):
        @pl.when(i % 2 == h)
        def _():
            wait(i, h)
            d = pb[h] - tb[h]
            d2 = (d * d).astype(jnp.bfloat16)
            mrow = m_ref[0].astype(jnp.bfloat16).reshape(1, _TB)
            part = jax.lax.dot_general(
                mrow, d2, (((1,), (0,)), ((), ())),
                preferred_element_type=jnp.float32)
            se_acc[0] += jnp.sum(part)
            n_acc[0] += jnp.sum(m_ref[0].astype(jnp.float32))

    @pl.when(i == _STEPS - 1)
    def _():
        se_ref[0] = se_acc[0]
        n_ref[0] = n_acc[0]


def _tc_call(predicted, target, m3):
    return pl.pallas_call(
        _tc_body,
        grid=(_STEPS,),
        in_specs=[
            pl.BlockSpec(memory_space=pltpu.MemorySpace.HBM),
            pl.BlockSpec(memory_space=pltpu.MemorySpace.HBM),
            pl.BlockSpec((1, _MR, 128),
                         lambda i: (i // _SLABS_PER_B, i % _SLABS_PER_B, 0)),
        ],
        out_specs=[
            pl.BlockSpec(memory_space=pltpu.MemorySpace.SMEM),
            pl.BlockSpec(memory_space=pltpu.MemorySpace.SMEM),
        ],
        out_shape=[
            jax.ShapeDtypeStruct((1,), jnp.float32),
            jax.ShapeDtypeStruct((1,), jnp.float32),
        ],
        scratch_shapes=[
            pltpu.VMEM((2, _TB, _C), jnp.float32),
            pltpu.VMEM((2, _TB, _C), jnp.float32),
            pltpu.SMEM((1,), jnp.float32),
            pltpu.SMEM((1,), jnp.float32),
            pltpu.SemaphoreType.DMA((2, _NQ)),
            pltpu.SemaphoreType.DMA((2, _NQ)),
        ],
        compiler_params=pltpu.CompilerParams(
            dimension_semantics=("arbitrary",),
            vmem_limit_bytes=60 * 1024 * 1024,
        ),
    )(predicted, target, m3)


_NC, _NS, _L = 2, 16, 16
_NW = _NC * _NS
_BSC = _B - _BT
_RPW = _BSC * _T // _NW
_BPT = _T // _RPW
_CHR = 64
_NCH = _RPW // _CHR
_GRP = _CHR // _L
_RING = 4
_MCOL = _RPW // 8


def _sc_body(p_hbm, t_hbm, m_hbm, se_hbm, n_hbm,
             mb0, mb1, mb2, mb3, pb0, pb1, pb2, pb3, tb0, tb1, tb2, tb3,
             acc_v, cnt_v,
             sm0, sm1, sm2, sm3, sp0, sp1, sp2, sp3, st0, st1, st2, st3):
    wid = lax.axis_index("s") * _NC + lax.axis_index("c")
    b = _BT + wid // _BPT
    t0 = (wid % _BPT) * _RPW

    acc_v[...] = jnp.zeros((_L,), jnp.float32)
    cnt_v[...] = jnp.zeros((_L,), jnp.int32)

    ones = jnp.full((_L,), 1, jnp.int32)

    def fire(ch, mb, pb, tb, sm, sp, st):
        r0 = t0 + ch * _CHR
        mr = wid * 8 + ch // (_MCOL // _CHR)
        mc = (ch % (_MCOL // _CHR)) * _CHR
        pltpu.async_copy(m_hbm.at[mr, pl.ds(mc, _CHR)], mb, sm)
        pltpu.async_copy(p_hbm.at[b, pl.ds(r0, _CHR)], pb, sp)
        pltpu.async_copy(t_hbm.at[b, pl.ds(r0, _CHR)], tb, st)

    def wait(ch, mb, pb, tb, sm, sp, st):
        r0 = t0 + ch * _CHR
        mr = wid * 8 + ch // (_MCOL // _CHR)
        mc = (ch % (_MCOL // _CHR)) * _CHR
        pltpu.make_async_copy(m_hbm.at[mr, pl.ds(mc, _CHR)], mb, sm).wait()
        pltpu.make_async_copy(p_hbm.at[b, pl.ds(r0, _CHR)], pb, sp).wait()
        pltpu.make_async_copy(t_hbm.at[b, pl.ds(r0, _CHR)], tb, st).wait()

    def compute(ch, mb, pb, tb):
        acc = jnp.zeros((_L,), jnp.float32)
        for g in range(_GRP):
            m16 = mb[pl.ds(g * _L, _L)]
            cnt_v[...] = cnt_v[...] + m16
            m16f = m16.astype(jnp.float32)
            jvec = jnp.zeros((_L,), jnp.int32)
            for j in range(_L):
                msp = lax.gather(
                    m16f, jvec[:, None],
                    lax.GatherDimensionNumbers(
                        offset_dims=(), collapsed_slice_dims=(0,),
                        start_index_map=(0,)),
                    (1,), mode=lax.GatherScatterMode.PROMISE_IN_BOUNDS)
                r = g * _L + j
                for c4 in range(_C // _L):
                    pv = pb[r, pl.ds(c4 * _L, _L)]
                    tv = tb[r, pl.ds(c4 * _L, _L)]
                    d = pv - tv
                    acc = acc + d * d * msp
                jvec = jvec + ones
        acc_v[...] = acc_v[...] + acc

    bufs = ((mb0, pb0, tb0, sm0, sp0, st0), (mb1, pb1, tb1, sm1, sp1, st1),
            (mb2, pb2, tb2, sm2, sp2, st2), (mb3, pb3, tb3, sm3, sp3, st3))

    for r in range(_RING):
        fire(r, *bufs[r])

    def ring(kk, _):
        ch = _RING * kk
        for h in range(_RING):
            cc = ch + h
            wait(cc, *bufs[h])
            compute(cc, bufs[h][0], bufs[h][1], bufs[h][2])

            @pl.when(cc + _RING < _NCH)
            def _():
                fire(cc + _RING, *bufs[h])

        return 0

    lax.fori_loop(0, _NCH // _RING, ring, 0)

    pltpu.sync_copy(acc_v, se_hbm.at[wid])
    pltpu.sync_copy(cnt_v, n_hbm.at[wid])


def _sc_call(p3, t3, mi):
    kfn = pl.kernel(
        _sc_body,
        out_type=[
            jax.ShapeDtypeStruct((_NW, _L), jnp.float32),
            jax.ShapeDtypeStruct((_NW, _L), jnp.int32),
        ],
        mesh=plsc.VectorSubcoreMesh(core_axis_name="c", subcore_axis_name="s"),
        scratch_types=(
            [pltpu.VMEM((_CHR,), jnp.int32) for _ in range(_RING)]
            + [pltpu.VMEM((_CHR, _C), jnp.float32) for _ in range(2 * _RING)]
            + [pltpu.VMEM((_L,), jnp.float32), pltpu.VMEM((_L,), jnp.int32)]
            + [pltpu.SemaphoreType.DMA for _ in range(3 * _RING)]
        ),
        compiler_params=pltpu.CompilerParams(needs_layout_passes=False),
    )
    return kfn(p3, t3, mi)


def kernel(predicted, target, mask):
    m3 = mask.reshape(_B, _T // 128, 128)
    mi = mask[_BT:].astype(jnp.int32).reshape(_NW * 8, _MCOL)

    se_sc, n_sc = _sc_call(predicted, target, mi)
    se_tc, n_tc = _tc_call(predicted, target, m3)

    se = se_tc[0] + jnp.sum(se_sc)
    n = n_tc[0] + jnp.sum(n_sc).astype(jnp.float32)
    count = n * jnp.float32(_C)
    safe = jnp.where(count == 0.0, jnp.float32(1.0), count)
    return jnp.where(n == 0.0, jnp.float32(0.0), se / safe)
